# baseline (speedup 1.0000x reference)
import jax
import jax.numpy as jnp
from jax import lax
from jax.experimental import pallas as pl
from jax.experimental.pallas import tpu as pltpu
from jax.experimental.pallas import tpu_sc as plsc

NFEAT = 1000000
NEMB = 64
B = 16384
F = 26

NC = 2
NS = 16
NW = NC * NS

BF = B * F
N_PER_W = BF // NW
G = 128
NG = N_PER_W // G
NBUF = 4
NSTEP = NG // NBUF
LANES = NEMB // 16


def _emb_body(table_hbm, idx_hbm, val_hbm, out_hbm,
              idx_v, val_v, rows_v, obuf_v, gsems, osems):
    wid = lax.axis_index("s") * NC + lax.axis_index("c")
    base = wid * N_PER_W

    pltpu.sync_copy(idx_hbm.at[pl.ds(base, N_PER_W)], idx_v)
    pltpu.sync_copy(val_hbm.at[pl.ds(base, N_PER_W)], val_v)

    def fire_gather(g, b):
        pltpu.make_async_copy(
            table_hbm.at[idx_v.at[pl.ds(g * G, G)]],
            rows_v.at[b],
            gsems.at[b],
        ).start()

    def wait_gather(b):
        pltpu.make_async_copy(
            table_hbm.at[idx_v.at[pl.ds(0, G)]],
            rows_v.at[b],
            gsems.at[b],
        ).wait()

    def start_store(g, b):
        pltpu.make_async_copy(
            obuf_v.at[b],
            out_hbm.at[pl.ds(base + g * G, G)],
            osems.at[b],
        ).start()

    def wait_store(b):
        pltpu.make_async_copy(
            obuf_v.at[b],
            out_hbm.at[pl.ds(base, G)],
            osems.at[b],
        ).wait()

    for b in range(NBUF):
        fire_gather(b, b)

    def step(t, _):
        for b in range(NBUF):
            g = t * NBUF + b
            wait_gather(b)

            @pl.when(t > 0)
            def _():
                wait_store(b)

            src = rows_v.at[b]
            dst = obuf_v.at[b]

            def blk(q, _):
                vv = val_v[pl.ds(g * G + q * 16, 16)]
                for r16 in range(16):
                    r = q * 16 + r16
                    v = vv[r16]
                    for j in range(LANES):
                        sl = pl.ds(j * 16, 16)
                        dst[r, sl] = src[r, sl] * v
                return 0

            lax.fori_loop(0, G // 16, blk, 0)

            @pl.when(g + NBUF < NG)
            def _():
                fire_gather(g + NBUF, b)

            start_store(g, b)
        return 0

    lax.fori_loop(0, NSTEP, step, 0)

    for b in range(NBUF):
        wait_store(b)


def _make_emb():
    mesh = plsc.VectorSubcoreMesh(core_axis_name="c", subcore_axis_name="s")
    return pl.kernel(
        _emb_body,
        out_type=jax.ShapeDtypeStruct((BF, NEMB), jnp.float32),
        mesh=mesh,
        compiler_params=pltpu.CompilerParams(use_tc_tiling_on_sc=False),
        scratch_types=[
            pltpu.VMEM((N_PER_W,), jnp.int32),
            pltpu.VMEM((N_PER_W,), jnp.float32),
            pltpu.VMEM((NBUF, G, NEMB), jnp.float32),
            pltpu.VMEM((NBUF, G, NEMB), jnp.float32),
            pltpu.SemaphoreType.DMA((NBUF,)),
            pltpu.SemaphoreType.DMA((NBUF,)),
        ],
    )


@jax.jit
def kernel(id, value, table):
    idx = id.reshape(BF)
    val = value.reshape(BF)
    out = _make_emb()(table, idx, val)
    return out.reshape(B, F, NEMB)
